# Initial kernel scaffold; baseline (speedup 1.0000x reference)
import jax
import jax.numpy as jnp
from jax.experimental import pallas as pl


def kernel(x, edge_index, W1, b1, W2, b2):
    raise NotImplementedError("write your pallas kernel here")


# speedup vs baseline: 5.3482x; 5.3482x over previous
import functools

import jax
import jax.numpy as jnp
from jax import lax
from jax.experimental import pallas as pl
from jax.experimental.pallas import tpu as pltpu
from jax.experimental.pallas import tpu_sc as plsc

N_NODES = 10000
N_EDGES = 320000
D = 128

NC = 2
NS = 16

EDGES_PER_TILE = N_EDGES // (NC * NS)
CHUNK = 80
NCHUNKS = EDGES_PER_TILE // CHUNK
ROWS_PER_TILE = 640


def _sc_aggregate(x, src, dst, zeros):
    mesh = plsc.VectorSubcoreMesh(core_axis_name="c", subcore_axis_name="s")

    @functools.partial(
        pl.kernel,
        mesh=mesh,
        out_type=jax.ShapeDtypeStruct((NC, N_NODES, D), jnp.float32),
        scratch_types=[
            pltpu.VMEM((CHUNK,), jnp.int32),
            pltpu.VMEM((CHUNK,), jnp.int32),
            pltpu.VMEM((CHUNK, D), jnp.float32),
            pltpu.VMEM_SHARED((N_NODES, D), jnp.float32),
            pltpu.SemaphoreType.DMA,
        ],
    )
    def agg(x_hbm, src_hbm, dst_hbm, zeros_hbm, out_hbm,
            src_v, dst_v, rows_v, aggr_sh, sem):
        c = lax.axis_index("c")
        s = lax.axis_index("s")

        row0 = pl.multiple_of(lax.min(s * ROWS_PER_TILE, N_NODES - ROWS_PER_TILE), 8)
        pltpu.sync_copy(zeros_hbm.at[pl.ds(row0, ROWS_PER_TILE)],
                        aggr_sh.at[pl.ds(row0, ROWS_PER_TILE)])
        plsc.subcore_barrier()

        tile_base = (c * NS + s) * EDGES_PER_TILE

        def body(i, carry):
            base = tile_base + i * CHUNK
            pltpu.sync_copy(src_hbm.at[pl.ds(base, CHUNK)], src_v)
            pltpu.sync_copy(dst_hbm.at[pl.ds(base, CHUNK)], dst_v)
            pltpu.async_copy(x_hbm.at[src_v], rows_v, sem).wait()
            pltpu.sync_copy(rows_v, aggr_sh.at[dst_v], add=True)
            return carry

        lax.fori_loop(0, NCHUNKS, body, 0)
        plsc.subcore_barrier()

        pltpu.sync_copy(aggr_sh.at[pl.ds(row0, ROWS_PER_TILE)],
                        out_hbm.at[c, pl.ds(row0, ROWS_PER_TILE)])

    return agg(x, src, dst, zeros)


def _mlp(x, p0, p1, W1, b1, W2, b2):
    BLK = 1000

    def body(x_ref, p0_ref, p1_ref, w1_ref, b1_ref, w2_ref, b2_ref, o_ref):
        h = x_ref[...] + p0_ref[...] + p1_ref[...]
        h = jnp.maximum(
            jnp.dot(h, w1_ref[...], preferred_element_type=jnp.float32)
            + b1_ref[...], 0.0)
        o_ref[...] = (
            jnp.dot(h, w2_ref[...], preferred_element_type=jnp.float32)
            + b2_ref[...])

    return pl.pallas_call(
        body,
        grid=(N_NODES // BLK,),
        in_specs=[
            pl.BlockSpec((BLK, D), lambda i: (i, 0)),
            pl.BlockSpec((BLK, D), lambda i: (i, 0)),
            pl.BlockSpec((BLK, D), lambda i: (i, 0)),
            pl.BlockSpec((D, D), lambda i: (0, 0)),
            pl.BlockSpec((1, D), lambda i: (0, 0)),
            pl.BlockSpec((D, D), lambda i: (0, 0)),
            pl.BlockSpec((1, D), lambda i: (0, 0)),
        ],
        out_specs=pl.BlockSpec((BLK, D), lambda i: (i, 0)),
        out_shape=jax.ShapeDtypeStruct((N_NODES, D), jnp.float32),
    )(x, p0, p1, W1, b1.reshape(1, D), W2, b2.reshape(1, D))


def kernel(x, edge_index, W1, b1, W2, b2):
    ei = edge_index.astype(jnp.int32)
    src = ei[0]
    dst = ei[1]
    zeros = jnp.zeros((N_NODES, D), jnp.float32)
    partials = _sc_aggregate(x, src, dst, zeros)
    return _mlp(x, partials[0], partials[1], W1, b1, W2, b2)


# speedup vs baseline: 11.1954x; 2.0933x over previous
import functools

import jax
import jax.numpy as jnp
from jax import lax
from jax.experimental import pallas as pl
from jax.experimental.pallas import tpu as pltpu
from jax.experimental.pallas import tpu_sc as plsc

N_NODES = 10000
N_EDGES = 320000
D = 128

NC = 2
NS = 16

EDGES_PER_TILE = N_EDGES // (NC * NS)
CHUNK = 80
NCHUNKS = EDGES_PER_TILE // CHUNK
ROWS_PER_TILE = 640


NVEC = CHUNK // 16


def _sc_aggregate(x, src, dst, zeros):
    mesh = plsc.VectorSubcoreMesh(core_axis_name="c", subcore_axis_name="s")

    @functools.partial(
        pl.kernel,
        mesh=mesh,
        out_type=jax.ShapeDtypeStruct((NC, N_NODES, D), jnp.float32),
        scratch_types=[
            pltpu.VMEM((EDGES_PER_TILE,), jnp.int32),
            pltpu.VMEM((EDGES_PER_TILE,), jnp.int32),
            pltpu.VMEM((CHUNK,), jnp.int32),
            pltpu.VMEM((CHUNK,), jnp.int32),
            pltpu.VMEM((CHUNK,), jnp.int32),
            pltpu.VMEM((CHUNK,), jnp.int32),
            pltpu.VMEM((CHUNK, D), jnp.float32),
            pltpu.VMEM((CHUNK, D), jnp.float32),
            pltpu.VMEM_SHARED((N_NODES, D), jnp.float32),
            pltpu.SemaphoreType.DMA,
            pltpu.SemaphoreType.DMA,
        ],
    )
    def agg(x_hbm, src_hbm, dst_hbm, zeros_hbm, out_hbm,
            src_all, dst_all, src_a, src_b, dst_a, dst_b,
            rows_a, rows_b, aggr_sh, sem_a, sem_b):
        c = lax.axis_index("c")
        s = lax.axis_index("s")

        row0 = pl.multiple_of(lax.min(s * ROWS_PER_TILE, N_NODES - ROWS_PER_TILE), 8)
        pltpu.sync_copy(zeros_hbm.at[pl.ds(row0, ROWS_PER_TILE)],
                        aggr_sh.at[pl.ds(row0, ROWS_PER_TILE)])

        tile_base = (c * NS + s) * EDGES_PER_TILE
        pltpu.sync_copy(src_hbm.at[pl.ds(tile_base, EDGES_PER_TILE)], src_all)
        pltpu.sync_copy(dst_hbm.at[pl.ds(tile_base, EDGES_PER_TILE)], dst_all)
        plsc.subcore_barrier()

        def fire(chunk, src_v, dst_v, rows_v, sem):
            base = chunk * CHUNK
            for k in range(NVEC):
                src_v[pl.ds(16 * k, 16)] = src_all[pl.ds(base + 16 * k, 16)]
                dst_v[pl.ds(16 * k, 16)] = dst_all[pl.ds(base + 16 * k, 16)]
            pltpu.async_copy(x_hbm.at[src_v], rows_v, sem)

        def drain(src_v, rows_v, sem):
            pltpu.make_async_copy(x_hbm.at[src_v], rows_v, sem).wait()

        fire(0, src_a, dst_a, rows_a, sem_a)

        def body(j, carry):
            fire(2 * j + 1, src_b, dst_b, rows_b, sem_b)
            drain(src_a, rows_a, sem_a)
            pltpu.sync_copy(rows_a, aggr_sh.at[dst_a], add=True)
            fire(2 * j + 2, src_a, dst_a, rows_a, sem_a)
            drain(src_b, rows_b, sem_b)
            pltpu.sync_copy(rows_b, aggr_sh.at[dst_b], add=True)
            return carry

        lax.fori_loop(0, (NCHUNKS - 1) // 2, body, 0)
        drain(src_a, rows_a, sem_a)
        pltpu.sync_copy(rows_a, aggr_sh.at[dst_a], add=True)
        plsc.subcore_barrier()

        pltpu.sync_copy(aggr_sh.at[pl.ds(row0, ROWS_PER_TILE)],
                        out_hbm.at[c, pl.ds(row0, ROWS_PER_TILE)])

    return agg(x, src, dst, zeros)


def _mlp(x, p0, p1, W1, b1, W2, b2):
    BLK = 1000

    def body(x_ref, p0_ref, p1_ref, w1_ref, b1_ref, w2_ref, b2_ref, o_ref):
        h = x_ref[...] + p0_ref[...] + p1_ref[...]
        h = jnp.maximum(
            jnp.dot(h, w1_ref[...], preferred_element_type=jnp.float32)
            + b1_ref[...], 0.0)
        o_ref[...] = (
            jnp.dot(h, w2_ref[...], preferred_element_type=jnp.float32)
            + b2_ref[...])

    return pl.pallas_call(
        body,
        grid=(N_NODES // BLK,),
        in_specs=[
            pl.BlockSpec((BLK, D), lambda i: (i, 0)),
            pl.BlockSpec((BLK, D), lambda i: (i, 0)),
            pl.BlockSpec((BLK, D), lambda i: (i, 0)),
            pl.BlockSpec((D, D), lambda i: (0, 0)),
            pl.BlockSpec((1, D), lambda i: (0, 0)),
            pl.BlockSpec((D, D), lambda i: (0, 0)),
            pl.BlockSpec((1, D), lambda i: (0, 0)),
        ],
        out_specs=pl.BlockSpec((BLK, D), lambda i: (i, 0)),
        out_shape=jax.ShapeDtypeStruct((N_NODES, D), jnp.float32),
    )(x, p0, p1, W1, b1.reshape(1, D), W2, b2.reshape(1, D))


def kernel(x, edge_index, W1, b1, W2, b2):
    ei = edge_index.astype(jnp.int32)
    src = ei[0]
    dst = ei[1]
    zeros = jnp.zeros((N_NODES, D), jnp.float32)
    partials = _sc_aggregate(x, src, dst, zeros)
    return _mlp(x, partials[0], partials[1], W1, b1, W2, b2)
